# baseline (speedup 1.0000x reference)
import jax
import jax.numpy as jnp
from jax import lax
from jax.experimental import pallas as pl
from jax.experimental.pallas import tpu as pltpu

DIM = 1024
TILE_N = 4096
NSPLIT = 4
SUB_N = TILE_N // NSPLIT
NUM_TOKENS = 256


def _fused_body(idx_ref, emb_hbm, *refs):
    w_refs = refs[:NSPLIT]
    b_ref = refs[NSPLIT]
    out_ref = refs[NSPLIT + 1]
    h_ref = refs[NSPLIT + 2]
    sem = refs[NSPLIT + 3]
    i = pl.program_id(0)

    @pl.when(i == 0)
    def _gather():
        def issue(t, _):
            pltpu.make_async_copy(
                emb_hbm.at[pl.ds(idx_ref[t], 1), :],
                h_ref.at[pl.ds(t, 1), :],
                sem,
            ).start()
            return 0

        def wait(t, _):
            pltpu.make_async_copy(
                emb_hbm.at[pl.ds(idx_ref[t], 1), :],
                h_ref.at[pl.ds(t, 1), :],
                sem,
            ).wait()
            return 0

        lax.fori_loop(0, NUM_TOKENS, issue, 0)
        lax.fori_loop(0, NUM_TOKENS, wait, 0)

    h = h_ref[...]
    for j in range(NSPLIT):
        acc = lax.dot_general(
            h, w_refs[j][...],
            dimension_numbers=(((1,), (1,)), ((), ())),
            preferred_element_type=jnp.float32,
        )
        sl = pl.ds(j * SUB_N, SUB_N)
        out_ref[:, sl] = acc + b_ref[:, sl]


@jax.jit
def kernel(x, emb_table, W, b):
    B, L = x.shape
    T = B * L
    V, D = W.shape
    idx = x.reshape(T).astype(jnp.int32)

    b2 = b.reshape(1, V)
    n_tiles = pl.cdiv(V, TILE_N)

    last_w_block = pl.cdiv(V, SUB_N) - 1

    def make_w_spec(j):
        return pl.BlockSpec(
            (SUB_N, D),
            lambda i, idx_ref, j=j: (jnp.minimum(NSPLIT * i + j, last_w_block), 0),
        )

    logits = pl.pallas_call(
        _fused_body,
        grid_spec=pltpu.PrefetchScalarGridSpec(
            num_scalar_prefetch=1,
            grid=(n_tiles,),
            in_specs=[pl.BlockSpec(memory_space=pl.ANY)]
            + [make_w_spec(j) for j in range(NSPLIT)]
            + [pl.BlockSpec((1, TILE_N), lambda i, idx_ref: (0, i))],
            out_specs=pl.BlockSpec((T, TILE_N), lambda i, idx_ref: (0, i)),
            scratch_shapes=[
                pltpu.VMEM((T, D), jnp.float32),
                pltpu.SemaphoreType.DMA,
            ],
        ),
        out_shape=jax.ShapeDtypeStruct((T, V), jnp.float32),
        compiler_params=pltpu.CompilerParams(
            dimension_semantics=("arbitrary",),
        ),
    )(idx, emb_table, W, W, W, W, b2)

    return logits.reshape(B, L, V)


# speedup vs baseline: 1.0080x; 1.0080x over previous
import jax
import jax.numpy as jnp
from jax import lax
from jax.experimental import pallas as pl
from jax.experimental.pallas import tpu as pltpu

DIM = 1024
TILE_N = 4096
NUM_TOKENS = 256


def _fused_body(idx_ref, emb_hbm, w_ref, b_ref, out_ref, h_ref, sem):
    i = pl.program_id(0)

    @pl.when(i == 0)
    def _gather():
        def issue(t, _):
            pltpu.make_async_copy(
                emb_hbm.at[pl.ds(idx_ref[t], 1), :],
                h_ref.at[pl.ds(t, 1), :],
                sem,
            ).start()
            return 0

        def wait(t, _):
            pltpu.make_async_copy(
                emb_hbm.at[pl.ds(idx_ref[t], 1), :],
                h_ref.at[pl.ds(t, 1), :],
                sem,
            ).wait()
            return 0

        lax.fori_loop(0, NUM_TOKENS, issue, 0)
        lax.fori_loop(0, NUM_TOKENS, wait, 0)

    @pl.when(i > 0)
    def _matmul():
        acc = lax.dot_general(
            h_ref[...], w_ref[...],
            dimension_numbers=(((1,), (1,)), ((), ())),
            preferred_element_type=jnp.float32,
        )
        out_ref[...] = acc + b_ref[...]


@jax.jit
def kernel(x, emb_table, W, b):
    B, L = x.shape
    T = B * L
    V, D = W.shape
    idx = x.reshape(T).astype(jnp.int32)

    b2 = b.reshape(1, V)
    n_tiles = pl.cdiv(V, TILE_N)

    def tile_ix(i, idx_ref):
        del idx_ref
        return (jnp.maximum(i - 1, 0), 0)

    def tile_ix_col(i, idx_ref):
        del idx_ref
        return (0, jnp.maximum(i - 1, 0))

    logits = pl.pallas_call(
        _fused_body,
        grid_spec=pltpu.PrefetchScalarGridSpec(
            num_scalar_prefetch=1,
            grid=(n_tiles + 1,),
            in_specs=[
                pl.BlockSpec(memory_space=pl.ANY),
                pl.BlockSpec((TILE_N, D), tile_ix),
                pl.BlockSpec((1, TILE_N), tile_ix_col),
            ],
            out_specs=pl.BlockSpec((T, TILE_N), tile_ix_col),
            scratch_shapes=[
                pltpu.VMEM((T, D), jnp.float32),
                pltpu.SemaphoreType.DMA,
            ],
        ),
        out_shape=jax.ShapeDtypeStruct((T, V), jnp.float32),
        compiler_params=pltpu.CompilerParams(
            dimension_semantics=("arbitrary",),
        ),
    )(idx, emb_table, W, b2)

    return logits.reshape(B, L, V)


# speedup vs baseline: 1.0214x; 1.0132x over previous
import jax
import jax.numpy as jnp
from jax import lax
from jax.experimental import pallas as pl
from jax.experimental.pallas import tpu as pltpu

DIM = 1024
TILE_N = 2048
DEPTH = 3
NUM_TOKENS = 256
VOCAB = 100000
N_FULL = VOCAB // TILE_N
REM = VOCAB - N_FULL * TILE_N
ISSUE_LAST_AT = N_FULL - DEPTH


def _body(idx_ref, emb_hbm, w_hbm, b_ref, out_hbm, h_ref, wbuf, obuf,
          wbuf_last, obuf_last, gsem, wsem, osem):
    def _w_fill(j_start, slot):
        return pltpu.make_async_copy(
            w_hbm.at[pl.ds(j_start, TILE_N), :], wbuf.at[slot], wsem.at[slot])

    def _w_fill_last():
        return pltpu.make_async_copy(
            w_hbm.at[pl.ds(N_FULL * TILE_N, REM), :], wbuf_last, gsem)

    def _out_copy(slot, col_start):
        return pltpu.make_async_copy(
            obuf.at[slot], out_hbm.at[:, pl.ds(col_start, TILE_N)],
            osem.at[slot])

    def issue_row(t, _):
        pltpu.make_async_copy(
            emb_hbm.at[pl.ds(idx_ref[t], 1), :],
            h_ref.at[pl.ds(t, 1), :], gsem).start()
        return 0

    def wait_row(t, _):
        pltpu.make_async_copy(
            emb_hbm.at[pl.ds(idx_ref[t], 1), :],
            h_ref.at[pl.ds(t, 1), :], gsem).wait()
        return 0

    lax.fori_loop(0, NUM_TOKENS, issue_row, 0)
    for j in range(DEPTH):
        _w_fill(j * TILE_N, j).start()
    lax.fori_loop(0, NUM_TOKENS, wait_row, 0)
    h = h_ref[...]

    def tile_step(i, _):
        slot = lax.rem(i, DEPTH)
        oslot = lax.rem(i, 2)
        s = i * TILE_N
        _w_fill(s, slot).wait()
        acc = lax.dot_general(
            h, wbuf[slot],
            dimension_numbers=(((1,), (1,)), ((), ())),
            preferred_element_type=jnp.float32,
        )

        @pl.when(i >= 2)
        def _reclaim():
            _out_copy(oslot, (i - 2) * TILE_N).wait()

        obuf[oslot] = acc + b_ref[:, pl.ds(s, TILE_N)]
        _out_copy(oslot, s).start()

        @pl.when(i + DEPTH < N_FULL)
        def _refill():
            _w_fill((i + DEPTH) * TILE_N, slot).start()

        @pl.when(i == ISSUE_LAST_AT)
        def _refill_last():
            _w_fill_last().start()

        return 0

    lax.fori_loop(0, N_FULL, tile_step, 0, unroll=2)

    _w_fill_last().wait()
    acc = lax.dot_general(
        h, wbuf_last[...],
        dimension_numbers=(((1,), (1,)), ((), ())),
        preferred_element_type=jnp.float32,
    )
    oslot = N_FULL % 2
    _out_copy(oslot, (N_FULL - 2) * TILE_N).wait()
    obuf_last[...] = acc + b_ref[:, pl.ds(N_FULL * TILE_N, REM)]
    last_out = pltpu.make_async_copy(
        obuf_last, out_hbm.at[:, pl.ds(N_FULL * TILE_N, REM)], osem.at[oslot])
    last_out.start()

    _out_copy(1 - oslot, (N_FULL - 1) * TILE_N).wait()
    last_out.wait()


@jax.jit
def kernel(x, emb_table, W, b):
    B, L = x.shape
    T = B * L
    V, D = W.shape
    idx = x.reshape(T).astype(jnp.int32)
    b2 = b.reshape(1, V)

    logits = pl.pallas_call(
        _body,
        grid_spec=pltpu.PrefetchScalarGridSpec(
            num_scalar_prefetch=1,
            grid=(1,),
            in_specs=[
                pl.BlockSpec(memory_space=pl.ANY),
                pl.BlockSpec(memory_space=pl.ANY),
                pl.BlockSpec((1, V), lambda i, idx_ref: (0, 0)),
            ],
            out_specs=pl.BlockSpec(memory_space=pl.ANY),
            scratch_shapes=[
                pltpu.VMEM((T, D), jnp.float32),
                pltpu.VMEM((DEPTH, TILE_N, D), jnp.float32),
                pltpu.VMEM((2, T, TILE_N), jnp.float32),
                pltpu.VMEM((REM, D), jnp.float32),
                pltpu.VMEM((T, REM), jnp.float32),
                pltpu.SemaphoreType.DMA,
                pltpu.SemaphoreType.DMA((DEPTH,)),
                pltpu.SemaphoreType.DMA((2,)),
            ],
        ),
        out_shape=jax.ShapeDtypeStruct((T, V), jnp.float32),
        compiler_params=pltpu.CompilerParams(
            dimension_semantics=("arbitrary",),
        ),
    )(idx, emb_table, W, b2)

    return logits.reshape(B, L, V)
